# Initial kernel scaffold; baseline (speedup 1.0000x reference)
import jax
import jax.numpy as jnp
from jax.experimental import pallas as pl


def kernel(token_ids, weight):
    raise NotImplementedError("write your pallas kernel here")


# speedup vs baseline: 1.2847x; 1.2847x over previous
import functools
import jax
import jax.numpy as jnp
from jax import lax
from jax.experimental import pallas as pl
from jax.experimental.pallas import tpu as pltpu
from jax.experimental.pallas import tpu_sc as plsc

NUM_EMB = 1000000
DIM = 32
TOTAL = 16384 * 50
CHUNK = 128
K = 8

_info = plsc.get_sparse_core_info()
NC, NS = _info.num_cores, _info.num_subcores
NW = NC * NS
K_TOT = TOTAL // (NW * CHUNK)
N_ITERS = K_TOT // K

_mesh = plsc.VectorSubcoreMesh(core_axis_name="c", subcore_axis_name="s")


@functools.partial(
    pl.kernel,
    mesh=_mesh,
    out_type=jax.ShapeDtypeStruct((NW, K_TOT, CHUNK, DIM), jnp.float32),
    scratch_types=[
        pltpu.VMEM((K, CHUNK), jnp.int32),
        pltpu.VMEM((K, CHUNK, DIM), jnp.float32),
        pltpu.SemaphoreType.DMA,
    ],
    compiler_params=pltpu.CompilerParams(use_tc_tiling_on_sc=False),
)
def _gather_kernel(idx_hbm, table_hbm, out_hbm, idx_v, rows_v, sem):
    wid = lax.axis_index("s") * NC + lax.axis_index("c")

    def body(g, carry):
        pltpu.sync_copy(idx_hbm.at[wid, pl.ds(g * K, K)], idx_v)
        copies = []
        for j in range(K):
            copies.append(
                pltpu.async_copy(table_hbm.at[idx_v.at[j]], rows_v.at[j], sem)
            )
        for c in copies:
            c.wait()
        pltpu.sync_copy(rows_v, out_hbm.at[wid, pl.ds(g * K, K)])
        return carry

    lax.fori_loop(0, N_ITERS, body, 0)


def kernel(token_ids, weight):
    idx = token_ids.reshape(NW, K_TOT, CHUNK).astype(jnp.int32)
    out = _gather_kernel(idx, weight)
    return out.reshape(16384, 50, DIM)


# speedup vs baseline: 1.3083x; 1.0184x over previous
import functools
import jax
import jax.numpy as jnp
from jax import lax
from jax.experimental import pallas as pl
from jax.experimental.pallas import tpu as pltpu
from jax.experimental.pallas import tpu_sc as plsc

NUM_EMB = 1000000
DIM = 32
TOTAL = 16384 * 50
CHUNK = 128
K = 10

_info = plsc.get_sparse_core_info()
NC, NS = _info.num_cores, _info.num_subcores
NW = NC * NS
K_TOT = TOTAL // (NW * CHUNK)
N_GROUPS = K_TOT // K
H = N_GROUPS // 2

_mesh = plsc.VectorSubcoreMesh(core_axis_name="c", subcore_axis_name="s")


@functools.partial(
    pl.kernel,
    mesh=_mesh,
    out_type=jax.ShapeDtypeStruct((NW, K_TOT, CHUNK, DIM), jnp.float32),
    scratch_types=[
        pltpu.VMEM((2, K, CHUNK), jnp.int32),
        pltpu.VMEM((2, K, CHUNK, DIM), jnp.float32),
        pltpu.SemaphoreType.DMA,
        pltpu.SemaphoreType.DMA,
        pltpu.SemaphoreType.DMA,
        pltpu.SemaphoreType.DMA,
    ],
    compiler_params=pltpu.CompilerParams(use_tc_tiling_on_sc=False),
)
def _gather_kernel(idx_hbm, table_hbm, out_hbm, idx_v, rows_v,
                   sem_g0, sem_g1, sem_o0, sem_o1):
    wid = lax.axis_index("s") * NC + lax.axis_index("c")

    def fire(buf, g, sem):
        for j in range(K):
            pltpu.async_copy(table_hbm.at[idx_v.at[buf, j]],
                             rows_v.at[buf, j], sem)

    def drain_gather(buf, sem):
        for j in range(K):
            pltpu.make_async_copy(table_hbm.at[idx_v.at[buf, j]],
                                  rows_v.at[buf, j], sem).wait()

    def store(buf, g, sem):
        pltpu.async_copy(rows_v.at[buf], out_hbm.at[wid, pl.ds(g * K, K)],
                         sem)

    def drain_store(buf, g, sem):
        pltpu.make_async_copy(rows_v.at[buf],
                              out_hbm.at[wid, pl.ds(g * K, K)], sem).wait()

    pltpu.sync_copy(idx_hbm.at[wid, pl.ds(0, K)], idx_v.at[0])
    fire(0, 0, sem_g0)

    def body(h, carry):
        a = 2 * h
        b = a + 1
        c = a + 2
        pltpu.sync_copy(idx_hbm.at[wid, pl.ds(b * K, K)], idx_v.at[1])
        fire(1, b, sem_g1)
        drain_gather(0, sem_g0)
        store(0, a, sem_o0)
        pltpu.sync_copy(idx_hbm.at[wid, pl.ds(c * K, K)], idx_v.at[0])
        drain_store(0, a, sem_o0)
        fire(0, c, sem_g0)
        drain_gather(1, sem_g1)
        store(1, b, sem_o1)
        drain_store(1, b, sem_o1)
        return carry

    lax.fori_loop(0, H - 1, body, 0)

    a = N_GROUPS - 2
    b = N_GROUPS - 1
    pltpu.sync_copy(idx_hbm.at[wid, pl.ds(b * K, K)], idx_v.at[1])
    fire(1, b, sem_g1)
    drain_gather(0, sem_g0)
    pltpu.sync_copy(rows_v.at[0], out_hbm.at[wid, pl.ds(a * K, K)])
    drain_gather(1, sem_g1)
    pltpu.sync_copy(rows_v.at[1], out_hbm.at[wid, pl.ds(b * K, K)])


def kernel(token_ids, weight):
    idx = token_ids.reshape(NW, K_TOT, CHUNK).astype(jnp.int32)
    out = _gather_kernel(idx, weight)
    return out.reshape(16384, 50, DIM)


# speedup vs baseline: 1.7729x; 1.3551x over previous
import functools
import jax
import jax.numpy as jnp
from jax import lax
from jax.experimental import pallas as pl
from jax.experimental.pallas import tpu as pltpu
from jax.experimental.pallas import tpu_sc as plsc

BATCH = 16384
HIST = 50
DIM = 32
R = 8

_info = plsc.get_sparse_core_info()
NC, NS = _info.num_cores, _info.num_subcores
NW = NC * NS
ROWS_PER_W = BATCH // NW
N_GROUPS = ROWS_PER_W // R
H = N_GROUPS // 2

_mesh = plsc.VectorSubcoreMesh(core_axis_name="c", subcore_axis_name="s")


@functools.partial(
    pl.kernel,
    mesh=_mesh,
    out_type=jax.ShapeDtypeStruct((BATCH, HIST, DIM), jnp.float32),
    scratch_types=[
        pltpu.VMEM((2, R, HIST), jnp.int32),
        pltpu.VMEM((2, R, HIST, DIM), jnp.float32),
        pltpu.SemaphoreType.DMA,
        pltpu.SemaphoreType.DMA,
        pltpu.SemaphoreType.DMA,
        pltpu.SemaphoreType.DMA,
    ],
    compiler_params=pltpu.CompilerParams(use_tc_tiling_on_sc=False),
)
def _gather_kernel(idx_hbm, table_hbm, out_hbm, idx_v, rows_v,
                   sem_g0, sem_g1, sem_o0, sem_o1):
    wid = lax.axis_index("s") * NC + lax.axis_index("c")
    base = wid * ROWS_PER_W

    def fire(buf, sem):
        for r in range(R):
            pltpu.async_copy(table_hbm.at[idx_v.at[buf, r]],
                             rows_v.at[buf, r], sem)

    def drain_gather(buf, sem):
        for r in range(R):
            pltpu.make_async_copy(table_hbm.at[idx_v.at[buf, r]],
                                  rows_v.at[buf, r], sem).wait()

    pltpu.sync_copy(idx_hbm.at[pl.ds(base, R)], idx_v.at[0])
    fire(0, sem_g0)

    def body(h, carry):
        row_a = base + (2 * h) * R
        row_b = row_a + R
        row_c = row_b + R
        pltpu.sync_copy(idx_hbm.at[pl.ds(row_b, R)], idx_v.at[1])
        fire(1, sem_g1)
        drain_gather(0, sem_g0)
        pltpu.async_copy(rows_v.at[0], out_hbm.at[pl.ds(row_a, R)], sem_o0)
        pltpu.sync_copy(idx_hbm.at[pl.ds(row_c, R)], idx_v.at[0])
        pltpu.make_async_copy(rows_v.at[0], out_hbm.at[pl.ds(row_a, R)],
                              sem_o0).wait()
        fire(0, sem_g0)
        drain_gather(1, sem_g1)
        pltpu.async_copy(rows_v.at[1], out_hbm.at[pl.ds(row_b, R)], sem_o1)
        pltpu.make_async_copy(rows_v.at[1], out_hbm.at[pl.ds(row_b, R)],
                              sem_o1).wait()
        return carry

    lax.fori_loop(0, H - 1, body, 0)

    row_a = base + (N_GROUPS - 2) * R
    row_b = row_a + R
    pltpu.sync_copy(idx_hbm.at[pl.ds(row_b, R)], idx_v.at[1])
    fire(1, sem_g1)
    drain_gather(0, sem_g0)
    pltpu.sync_copy(rows_v.at[0], out_hbm.at[pl.ds(row_a, R)])
    drain_gather(1, sem_g1)
    pltpu.sync_copy(rows_v.at[1], out_hbm.at[pl.ds(row_b, R)])


def kernel(token_ids, weight):
    return _gather_kernel(token_ids.astype(jnp.int32), weight)
